# baseline (speedup 1.0000x reference)
import jax
import jax.numpy as jnp
from jax import lax
from jax.experimental import pallas as pl
from jax.experimental.pallas import tpu as pltpu
from jax.experimental.pallas import tpu_sc as plsc

N = 10000
E = 320000
D_IN = 128
HEADS = 8
HID = 16
OUT = 16

NC = 2
NS = 16
NW = NC * NS
EPT = E // NW
B = 80
NB = EPT // B
NP = 10240
NPT = NP // NS

_f32 = jnp.float32


def _sds(shape):
    return jax.ShapeDtypeStruct(shape, _f32)


def _k1_body(x_ref, w1_ref, a1s_ref, a1d_ref, h1_ref, as_ref, ad_ref):
    x = x_ref[...]
    h1 = jnp.dot(x, w1_ref[...], preferred_element_type=_f32)
    h1_ref[...] = h1
    d_idx = lax.broadcasted_iota(jnp.int32, (D_IN, 16), 0)
    h_idx = lax.broadcasted_iota(jnp.int32, (D_IN, 16), 1)
    s = (d_idx // HID == h_idx).astype(_f32)
    as_ref[...] = jnp.dot(h1 * a1s_ref[...], s, preferred_element_type=_f32)
    ad_ref[...] = jnp.dot(h1 * a1d_ref[...], s, preferred_element_type=_f32)


def _k1(x, w1, a1s, a1d):
    return pl.pallas_call(
        _k1_body,
        out_shape=[_sds((N, D_IN)), _sds((N, 16)), _sds((N, 16))],
    )(x, w1, a1s, a1d)


def _edge_kernel(row_w):
    def body(h_hbm, as_hbm, ad_hbm, src_hbm, dst_hbm, zrow_hbm, zw_hbm,
             ph_hbm, pw_hbm,
             idx_s0, idx_d0, asb0, adb0, hb0,
             idx_s1, idx_d1, asb1, adb1, hb1,
             idx_c, wb, mb, acc_h, acc_w, sg0, sg1, ss):
        c = lax.axis_index("c")
        s = lax.axis_index("s")
        wid = s * NC + c
        ebase = wid * EPT
        nbase = s * NPT

        bufs = ((idx_s0, idx_d0, asb0, adb0, hb0, sg0),
                (idx_s1, idx_d1, asb1, adb1, hb1, sg1))

        pltpu.sync_copy(zrow_hbm.at[pl.ds(nbase, NPT)],
                        acc_h.at[pl.ds(nbase, NPT)])
        pltpu.sync_copy(zw_hbm.at[pl.ds(nbase, NPT)],
                        acc_w.at[pl.ds(nbase, NPT)])
        plsc.subcore_barrier()

        def issue(b, p):
            idx_s, idx_d, asb, adb, hb, sg = bufs[p]
            base = ebase + b * B
            pltpu.sync_copy(src_hbm.at[pl.ds(base, B)], idx_s)
            pltpu.sync_copy(dst_hbm.at[pl.ds(base, B)], idx_d)
            pltpu.async_copy(as_hbm.at[idx_s], asb, sg)
            pltpu.async_copy(ad_hbm.at[idx_d], adb, sg)
            pltpu.async_copy(h_hbm.at[idx_s], hb, sg)

        def stage(b, p, last):
            idx_s, idx_d, asb, adb, hb, sg = bufs[p]
            pltpu.make_async_copy(as_hbm.at[idx_s], asb, sg).wait()
            pltpu.make_async_copy(ad_hbm.at[idx_d], adb, sg).wait()
            pltpu.make_async_copy(h_hbm.at[idx_s], hb, sg).wait()
            @pl.when(b >= 1)
            def _():
                pltpu.make_async_copy(mb, acc_h.at[idx_c], ss).wait()
                pltpu.make_async_copy(wb, acc_w.at[idx_c], ss).wait()
            for k in range(B // 16):
                idx_c[pl.ds(k * 16, 16)] = idx_d[pl.ds(k * 16, 16)]

            @plsc.parallel_loop(0, B, 1, unroll=8)
            def _(e):
                v = asb[e, :] + adb[e, :]
                v = jnp.where(v >= 0.0, v, 0.2 * v)
                w_v = jnp.exp(v)
                wb[e, :] = w_v
                for h in range(row_w // 16):
                    mb[e, pl.ds(h * 16, 16)] = (
                        hb[e, pl.ds(h * 16, 16)]
                        * jnp.full((16,), w_v[h], _f32))
            c1 = pltpu.async_copy(mb, acc_h.at[idx_c], ss, add=True)
            c2 = pltpu.async_copy(wb, acc_w.at[idx_c], ss, add=True)
            if last:
                c1.wait()
                c2.wait()
            else:
                @pl.when(b + 2 < NB)
                def _():
                    issue(b + 2, p)

        issue(0, 0)
        issue(1, 1)

        def pair(i, carry):
            stage(2 * i, 0, False)
            stage(2 * i + 1, 1, False)
            return carry

        lax.fori_loop(0, NB // 2, pair, 0)
        stage(NB - 1, 0, True)
        plsc.subcore_barrier()

        pltpu.sync_copy(acc_h.at[pl.ds(nbase, NPT)],
                        ph_hbm.at[c, pl.ds(nbase, NPT)])
        pltpu.sync_copy(acc_w.at[pl.ds(nbase, NPT)],
                        pw_hbm.at[c, pl.ds(nbase, NPT)])

    return body


_edge_call_cache = {}


def _make_edge_call(row_w):
    if row_w in _edge_call_cache:
        return _edge_call_cache[row_w]
    mesh = plsc.VectorSubcoreMesh(core_axis_name="c", subcore_axis_name="s",
                                  num_cores=NC, num_subcores=NS)
    call = pl.kernel(
        _edge_kernel(row_w),
        out_type=[_sds((NC, NP, row_w)), _sds((NC, NP, 16))],
        mesh=mesh,
        compiler_params=pltpu.CompilerParams(use_tc_tiling_on_sc=False),
        scratch_types=(
            [pltpu.VMEM((B,), jnp.int32),
             pltpu.VMEM((B,), jnp.int32),
             pltpu.VMEM((B, 16), _f32),
             pltpu.VMEM((B, 16), _f32),
             pltpu.VMEM((B, row_w), _f32),
             ] * 2
            + [pltpu.VMEM((B,), jnp.int32),
               pltpu.VMEM((B, 16), _f32),
               pltpu.VMEM((B, row_w), _f32),
               pltpu.VMEM_SHARED((NP, row_w), _f32),
               pltpu.VMEM_SHARED((NP, 16), _f32),
               pltpu.SemaphoreType.DMA,
               pltpu.SemaphoreType.DMA,
               pltpu.SemaphoreType.DMA,
               ]),
    )
    _edge_call_cache[row_w] = call
    return call


def _k3_body(ph_ref, pw_ref, w2_ref, a2s_ref, a2d_ref,
             h2_ref, as2_ref, ad2_ref):
    acc = ph_ref[0] + ph_ref[1]
    den = pw_ref[0] + pw_ref[1]
    h_idx = lax.broadcasted_iota(jnp.int32, (16, D_IN), 0)
    d_idx = lax.broadcasted_iota(jnp.int32, (16, D_IN), 1)
    st = (d_idx // HID == h_idx).astype(_f32)
    den_rep = jnp.dot(den, st, preferred_element_type=_f32)
    x2 = acc / (den_rep + 1e-16)
    x2 = jnp.where(x2 > 0.0, x2, jnp.exp(jnp.minimum(x2, 0.0)) - 1.0)
    h2 = jnp.dot(x2, w2_ref[...], preferred_element_type=_f32)
    h2_ref[...] = h2
    j_idx = lax.broadcasted_iota(jnp.int32, (OUT, 16), 1)
    p0 = (j_idx == 0).astype(_f32)
    as2_ref[...] = jnp.dot(h2 * a2s_ref[...], p0, preferred_element_type=_f32)
    ad2_ref[...] = jnp.dot(h2 * a2d_ref[...], p0, preferred_element_type=_f32)


def _k3(ph, pw, w2, a2s, a2d):
    return pl.pallas_call(
        _k3_body,
        out_shape=[_sds((NP, OUT)), _sds((NP, 16)), _sds((NP, 16))],
    )(ph, pw, w2, a2s, a2d)


def _k5_body(ph_ref, pw_ref, out_ref):
    acc = ph_ref[0] + ph_ref[1]
    den = pw_ref[0, :, 0:1] + pw_ref[1, :, 0:1]
    out_ref[...] = acc / (den + 1e-16)


def _k5(ph, pw):
    return pl.pallas_call(_k5_body, out_shape=_sds((NP, OUT)))(ph, pw)


def kernel(x, edge_index, W1, a1_src, a1_dst, W2, a2_src, a2_dst):
    src = edge_index[0].astype(jnp.int32)
    dst = edge_index[1].astype(jnp.int32)
    a1s = a1_src.reshape(1, HEADS * HID)
    a1d = a1_dst.reshape(1, HEADS * HID)
    a2s = a2_src.reshape(1, OUT)
    a2d = a2_dst.reshape(1, OUT)
    zrow = jnp.zeros((NP, D_IN), _f32)
    z16 = jnp.zeros((NP, 16), _f32)

    h1, as1, ad1 = _k1(x, W1, a1s, a1d)
    ph1, pw1 = _make_edge_call(D_IN)(h1, as1, ad1, src, dst, zrow, z16)
    h2, as2, ad2 = _k3(ph1, pw1, W2, a2s, a2d)
    ph2, pw2 = _make_edge_call(OUT)(h2, as2, ad2, src, dst, z16, z16)
    return _k5(ph2, pw2)[:N]


# speedup vs baseline: 1.2610x; 1.2610x over previous
import jax
import jax.numpy as jnp
from jax import lax
from jax.experimental import pallas as pl
from jax.experimental.pallas import tpu as pltpu
from jax.experimental.pallas import tpu_sc as plsc

N = 10000
E = 320000
D_IN = 128
HEADS = 8
HID = 16
OUT = 16

NC = 2
NS = 16
NW = NC * NS
EPT = E // NW
B = 80
NB = EPT // B
NP = 10240
NPT = NP // NS

_f32 = jnp.float32


def _sds(shape):
    return jax.ShapeDtypeStruct(shape, _f32)


def _k1_body(x_ref, w1_ref, a1s_ref, a1d_ref, h1_ref, as_ref, ad_ref):
    x = x_ref[...]
    h1 = jnp.dot(x, w1_ref[...], preferred_element_type=_f32)
    h1_ref[...] = h1
    d_idx = lax.broadcasted_iota(jnp.int32, (D_IN, 16), 0)
    h_idx = lax.broadcasted_iota(jnp.int32, (D_IN, 16), 1)
    s = (d_idx // HID == h_idx).astype(_f32)
    as_ref[...] = jnp.dot(h1 * a1s_ref[...], s, preferred_element_type=_f32)
    ad_ref[...] = jnp.dot(h1 * a1d_ref[...], s, preferred_element_type=_f32)


def _k1(x, w1, a1s, a1d):
    return pl.pallas_call(
        _k1_body,
        out_shape=[_sds((N, D_IN)), _sds((N, 16)), _sds((N, 16))],
    )(x, w1, a1s, a1d)


def _edge_kernel(row_w):
    def body(h_hbm, as_hbm, ad_hbm, src_hbm, dst_hbm, zrow_hbm, zw_hbm,
             ph_hbm, pw_hbm,
             idx_s0, idx_d0, asb0, adb0, hb0,
             idx_s1, idx_d1, asb1, adb1, hb1,
             idx_c, wb, mb, acc_h, acc_w, sg0, sg1, si0, si1, ss):
        c = lax.axis_index("c")
        s = lax.axis_index("s")
        wid = s * NC + c
        ebase = wid * EPT
        nbase = s * NPT

        bufs = ((idx_s0, idx_d0, asb0, adb0, hb0, sg0, si0),
                (idx_s1, idx_d1, asb1, adb1, hb1, sg1, si1))

        pltpu.sync_copy(zrow_hbm.at[pl.ds(nbase, NPT)],
                        acc_h.at[pl.ds(nbase, NPT)])
        pltpu.sync_copy(zw_hbm.at[pl.ds(nbase, NPT)],
                        acc_w.at[pl.ds(nbase, NPT)])
        plsc.subcore_barrier()

        def issue(b, p):
            idx_s, idx_d, asb, adb, hb, sg, si = bufs[p]
            base = ebase + b * B
            pltpu.sync_copy(src_hbm.at[pl.ds(base, B)], idx_s)
            pltpu.sync_copy(dst_hbm.at[pl.ds(base, B)], idx_d)
            pltpu.async_copy(as_hbm.at[idx_s], asb, sg)
            pltpu.async_copy(ad_hbm.at[idx_d], adb, sg)
            pltpu.async_copy(h_hbm.at[idx_s], hb, sg)

        def stage(b, p, last):
            idx_s, idx_d, asb, adb, hb, sg, si = bufs[p]
            pltpu.make_async_copy(as_hbm.at[idx_s], asb, sg).wait()
            pltpu.make_async_copy(ad_hbm.at[idx_d], adb, sg).wait()
            pltpu.make_async_copy(h_hbm.at[idx_s], hb, sg).wait()
            @pl.when(b >= 1)
            def _():
                pltpu.make_async_copy(mb, acc_h.at[idx_c], ss).wait()
                pltpu.make_async_copy(wb, acc_w.at[idx_c], ss).wait()
            for k in range(B // 16):
                idx_c[pl.ds(k * 16, 16)] = idx_d[pl.ds(k * 16, 16)]
            base2 = ebase + (b + 2) * B
            if not last:
                @pl.when(b + 2 < NB)
                def _():
                    pltpu.async_copy(src_hbm.at[pl.ds(base2, B)], idx_s, si)
                    pltpu.async_copy(dst_hbm.at[pl.ds(base2, B)], idx_d, si)

            @plsc.parallel_loop(0, B, 1, unroll=4)
            def _(e):
                v = asb[e, :] + adb[e, :]
                v = jnp.where(v >= 0.0, v, 0.2 * v)
                w_v = jnp.exp(v)
                wb[e, :] = w_v
                for h in range(row_w // 16):
                    mb[e, pl.ds(h * 16, 16)] = (
                        hb[e, pl.ds(h * 16, 16)]
                        * jnp.full((16,), w_v[h], _f32))
            c1 = pltpu.async_copy(mb, acc_h.at[idx_c], ss, add=True)
            c2 = pltpu.async_copy(wb, acc_w.at[idx_c], ss, add=True)
            if last:
                c1.wait()
                c2.wait()
            else:
                @pl.when(b + 2 < NB)
                def _():
                    pltpu.make_async_copy(
                        src_hbm.at[pl.ds(base2, B)], idx_s, si).wait()
                    pltpu.make_async_copy(
                        dst_hbm.at[pl.ds(base2, B)], idx_d, si).wait()
                    pltpu.async_copy(as_hbm.at[idx_s], asb, sg)
                    pltpu.async_copy(ad_hbm.at[idx_d], adb, sg)
                    pltpu.async_copy(h_hbm.at[idx_s], hb, sg)

        issue(0, 0)
        issue(1, 1)

        def pair(i, carry):
            stage(2 * i, 0, False)
            stage(2 * i + 1, 1, False)
            return carry

        lax.fori_loop(0, NB // 2, pair, 0)
        stage(NB - 1, 0, True)
        plsc.subcore_barrier()

        pltpu.sync_copy(acc_h.at[pl.ds(nbase, NPT)],
                        ph_hbm.at[c, pl.ds(nbase, NPT)])
        pltpu.sync_copy(acc_w.at[pl.ds(nbase, NPT)],
                        pw_hbm.at[c, pl.ds(nbase, NPT)])

    return body


_edge_call_cache = {}


def _make_edge_call(row_w):
    if row_w in _edge_call_cache:
        return _edge_call_cache[row_w]
    mesh = plsc.VectorSubcoreMesh(core_axis_name="c", subcore_axis_name="s",
                                  num_cores=NC, num_subcores=NS)
    call = pl.kernel(
        _edge_kernel(row_w),
        out_type=[_sds((NC, NP, row_w)), _sds((NC, NP, 16))],
        mesh=mesh,
        compiler_params=pltpu.CompilerParams(use_tc_tiling_on_sc=False),
        scratch_types=(
            [pltpu.VMEM((B,), jnp.int32),
             pltpu.VMEM((B,), jnp.int32),
             pltpu.VMEM((B, 16), _f32),
             pltpu.VMEM((B, 16), _f32),
             pltpu.VMEM((B, row_w), _f32),
             ] * 2
            + [pltpu.VMEM((B,), jnp.int32),
               pltpu.VMEM((B, 16), _f32),
               pltpu.VMEM((B, row_w), _f32),
               pltpu.VMEM_SHARED((NP, row_w), _f32),
               pltpu.VMEM_SHARED((NP, 16), _f32),
               pltpu.SemaphoreType.DMA,
               pltpu.SemaphoreType.DMA,
               pltpu.SemaphoreType.DMA,
               pltpu.SemaphoreType.DMA,
               pltpu.SemaphoreType.DMA,
               ]),
    )
    _edge_call_cache[row_w] = call
    return call


def _k3_body(ph_ref, pw_ref, w2_ref, a2s_ref, a2d_ref,
             h2_ref, as2_ref, ad2_ref):
    acc = ph_ref[0] + ph_ref[1]
    den = pw_ref[0] + pw_ref[1]
    h_idx = lax.broadcasted_iota(jnp.int32, (16, D_IN), 0)
    d_idx = lax.broadcasted_iota(jnp.int32, (16, D_IN), 1)
    st = (d_idx // HID == h_idx).astype(_f32)
    den_rep = jnp.dot(den, st, preferred_element_type=_f32)
    x2 = acc / (den_rep + 1e-16)
    x2 = jnp.where(x2 > 0.0, x2, jnp.exp(jnp.minimum(x2, 0.0)) - 1.0)
    h2 = jnp.dot(x2, w2_ref[...], preferred_element_type=_f32)
    h2_ref[...] = h2
    j_idx = lax.broadcasted_iota(jnp.int32, (OUT, 16), 1)
    p0 = (j_idx == 0).astype(_f32)
    as2_ref[...] = jnp.dot(h2 * a2s_ref[...], p0, preferred_element_type=_f32)
    ad2_ref[...] = jnp.dot(h2 * a2d_ref[...], p0, preferred_element_type=_f32)


def _k3(ph, pw, w2, a2s, a2d):
    return pl.pallas_call(
        _k3_body,
        out_shape=[_sds((NP, OUT)), _sds((NP, 16)), _sds((NP, 16))],
    )(ph, pw, w2, a2s, a2d)


def _k5_body(ph_ref, pw_ref, out_ref):
    acc = ph_ref[0] + ph_ref[1]
    den = pw_ref[0, :, 0:1] + pw_ref[1, :, 0:1]
    out_ref[...] = acc / (den + 1e-16)


def _k5(ph, pw):
    return pl.pallas_call(_k5_body, out_shape=_sds((NP, OUT)))(ph, pw)


def kernel(x, edge_index, W1, a1_src, a1_dst, W2, a2_src, a2_dst):
    src = edge_index[0].astype(jnp.int32)
    dst = edge_index[1].astype(jnp.int32)
    a1s = a1_src.reshape(1, HEADS * HID)
    a1d = a1_dst.reshape(1, HEADS * HID)
    a2s = a2_src.reshape(1, OUT)
    a2d = a2_dst.reshape(1, OUT)
    zrow = jnp.zeros((NP, D_IN), _f32)
    z16 = jnp.zeros((NP, 16), _f32)

    h1, as1, ad1 = _k1(x, W1, a1s, a1d)
    ph1, pw1 = _make_edge_call(D_IN)(h1, as1, ad1, src, dst, zrow, z16)
    h2, as2, ad2 = _k3(ph1, pw1, W2, a2s, a2d)
    ph2, pw2 = _make_edge_call(OUT)(h2, as2, ad2, src, dst, z16, z16)
    return _k5(ph2, pw2)[:N]


# speedup vs baseline: 1.3203x; 1.0470x over previous
import jax
import jax.numpy as jnp
from jax import lax
from jax.experimental import pallas as pl
from jax.experimental.pallas import tpu as pltpu
from jax.experimental.pallas import tpu_sc as plsc

N = 10000
E = 320000
D_IN = 128
HEADS = 8
HID = 16
OUT = 16

NC = 2
NS = 16
NW = NC * NS
EPT = E // NW
B = 80
NB = EPT // B
NP = 10240
NPT = NP // NS

_f32 = jnp.float32


def _sds(shape):
    return jax.ShapeDtypeStruct(shape, _f32)


def _k1_body(x_ref, w1_ref, a1s_ref, a1d_ref, h1_ref, as_ref, ad_ref):
    x = x_ref[...]
    h1 = jnp.dot(x, w1_ref[...], preferred_element_type=_f32)
    h1_ref[...] = h1
    d_idx = lax.broadcasted_iota(jnp.int32, (D_IN, 16), 0)
    h_idx = lax.broadcasted_iota(jnp.int32, (D_IN, 16), 1)
    s = (d_idx // HID == h_idx).astype(_f32)
    as_ref[...] = jnp.dot(h1 * a1s_ref[...], s, preferred_element_type=_f32)
    ad_ref[...] = jnp.dot(h1 * a1d_ref[...], s, preferred_element_type=_f32)


def _k1(x, w1, a1s, a1d):
    return pl.pallas_call(
        _k1_body,
        out_shape=[_sds((N, D_IN)), _sds((N, 16)), _sds((N, 16))],
    )(x, w1, a1s, a1d)


def _edge_kernel_preload(row_w):
    def body(h_hbm, as_hbm, ad_hbm, src_hbm, dst_hbm, zrow_hbm, zw_hbm,
             ph_hbm, pw_hbm,
             ixa_s, ixa_d, asb0, adb0, hb0, asb1, adb1, hb1,
             idx_c, wb, mb, acc_h, acc_w, sg0, sg1, ss):
        c = lax.axis_index("c")
        s = lax.axis_index("s")
        wid = s * NC + c
        ebase = wid * EPT
        nbase = s * NPT

        bufs = ((asb0, adb0, hb0, sg0), (asb1, adb1, hb1, sg1))

        pltpu.sync_copy(src_hbm.at[pl.ds(ebase, EPT)], ixa_s)
        pltpu.sync_copy(dst_hbm.at[pl.ds(ebase, EPT)], ixa_d)
        pltpu.sync_copy(zrow_hbm.at[pl.ds(nbase, NPT)],
                        acc_h.at[pl.ds(nbase, NPT)])
        pltpu.sync_copy(zw_hbm.at[pl.ds(nbase, NPT)],
                        acc_w.at[pl.ds(nbase, NPT)])
        plsc.subcore_barrier()

        def issue(b, p):
            asb, adb, hb, sg = bufs[p]
            off = b * B
            pltpu.async_copy(as_hbm.at[ixa_s.at[pl.ds(off, B)]], asb, sg)
            pltpu.async_copy(ad_hbm.at[ixa_d.at[pl.ds(off, B)]], adb, sg)
            pltpu.async_copy(h_hbm.at[ixa_s.at[pl.ds(off, B)]], hb, sg)

        def stage(b, p, last):
            asb, adb, hb, sg = bufs[p]
            off = b * B
            pltpu.make_async_copy(
                as_hbm.at[ixa_s.at[pl.ds(off, B)]], asb, sg).wait()
            pltpu.make_async_copy(
                ad_hbm.at[ixa_d.at[pl.ds(off, B)]], adb, sg).wait()
            pltpu.make_async_copy(
                h_hbm.at[ixa_s.at[pl.ds(off, B)]], hb, sg).wait()
            @pl.when(b >= 1)
            def _():
                pltpu.make_async_copy(mb, acc_h.at[idx_c], ss).wait()
                pltpu.make_async_copy(wb, acc_w.at[idx_c], ss).wait()
            for k in range(B // 16):
                idx_c[pl.ds(k * 16, 16)] = ixa_d[pl.ds(off + k * 16, 16)]

            @plsc.parallel_loop(0, B, 1, unroll=4)
            def _(e):
                v = asb[e, :] + adb[e, :]
                v = jnp.where(v >= 0.0, v, 0.2 * v)
                w_v = jnp.exp(v)
                wb[e, :] = w_v
                for h in range(row_w // 16):
                    mb[e, pl.ds(h * 16, 16)] = (
                        hb[e, pl.ds(h * 16, 16)]
                        * jnp.full((16,), w_v[h], _f32))

            c1 = pltpu.async_copy(mb, acc_h.at[idx_c], ss, add=True)
            c2 = pltpu.async_copy(wb, acc_w.at[idx_c], ss, add=True)
            if last:
                c1.wait()
                c2.wait()
            else:
                @pl.when(b + 2 < NB)
                def _():
                    issue(b + 2, p)

        issue(0, 0)
        issue(1, 1)

        def pair(i, carry):
            stage(2 * i, 0, False)
            stage(2 * i + 1, 1, False)
            return carry

        lax.fori_loop(0, NB // 2, pair, 0)
        stage(NB - 1, 0, True)
        plsc.subcore_barrier()

        pltpu.sync_copy(acc_h.at[pl.ds(nbase, NPT)],
                        ph_hbm.at[c, pl.ds(nbase, NPT)])
        pltpu.sync_copy(acc_w.at[pl.ds(nbase, NPT)],
                        pw_hbm.at[c, pl.ds(nbase, NPT)])

    return body


def _make_edge_call_preload(row_w):
    key = ("preload", row_w)
    if key in _edge_call_cache:
        return _edge_call_cache[key]
    mesh = plsc.VectorSubcoreMesh(core_axis_name="c", subcore_axis_name="s",
                                  num_cores=NC, num_subcores=NS)
    call = pl.kernel(
        _edge_kernel_preload(row_w),
        out_type=[_sds((NC, NP, row_w)), _sds((NC, NP, 16))],
        mesh=mesh,
        compiler_params=pltpu.CompilerParams(use_tc_tiling_on_sc=False),
        scratch_types=(
            [pltpu.VMEM((EPT,), jnp.int32),
             pltpu.VMEM((EPT,), jnp.int32)]
            + [pltpu.VMEM((B, 16), _f32),
               pltpu.VMEM((B, 16), _f32),
               pltpu.VMEM((B, row_w), _f32),
               ] * 2
            + [pltpu.VMEM((B,), jnp.int32),
               pltpu.VMEM((B, 16), _f32),
               pltpu.VMEM((B, row_w), _f32),
               pltpu.VMEM_SHARED((NP, row_w), _f32),
               pltpu.VMEM_SHARED((NP, 16), _f32),
               pltpu.SemaphoreType.DMA,
               pltpu.SemaphoreType.DMA,
               pltpu.SemaphoreType.DMA,
               ]),
    )
    _edge_call_cache[key] = call
    return call


def _edge_kernel(row_w):
    def body(h_hbm, as_hbm, ad_hbm, src_hbm, dst_hbm, zrow_hbm, zw_hbm,
             ph_hbm, pw_hbm,
             idx_s0, idx_d0, asb0, adb0, hb0,
             idx_s1, idx_d1, asb1, adb1, hb1,
             idx_c, wb, mb, acc_h, acc_w, sg0, sg1, si0, si1, ss):
        c = lax.axis_index("c")
        s = lax.axis_index("s")
        wid = s * NC + c
        ebase = wid * EPT
        nbase = s * NPT

        bufs = ((idx_s0, idx_d0, asb0, adb0, hb0, sg0, si0),
                (idx_s1, idx_d1, asb1, adb1, hb1, sg1, si1))

        pltpu.sync_copy(zrow_hbm.at[pl.ds(nbase, NPT)],
                        acc_h.at[pl.ds(nbase, NPT)])
        pltpu.sync_copy(zw_hbm.at[pl.ds(nbase, NPT)],
                        acc_w.at[pl.ds(nbase, NPT)])
        plsc.subcore_barrier()

        def issue(b, p):
            idx_s, idx_d, asb, adb, hb, sg, si = bufs[p]
            base = ebase + b * B
            pltpu.sync_copy(src_hbm.at[pl.ds(base, B)], idx_s)
            pltpu.sync_copy(dst_hbm.at[pl.ds(base, B)], idx_d)
            pltpu.async_copy(as_hbm.at[idx_s], asb, sg)
            pltpu.async_copy(ad_hbm.at[idx_d], adb, sg)
            pltpu.async_copy(h_hbm.at[idx_s], hb, sg)

        def stage(b, p, last):
            idx_s, idx_d, asb, adb, hb, sg, si = bufs[p]
            pltpu.make_async_copy(as_hbm.at[idx_s], asb, sg).wait()
            pltpu.make_async_copy(ad_hbm.at[idx_d], adb, sg).wait()
            pltpu.make_async_copy(h_hbm.at[idx_s], hb, sg).wait()
            @pl.when(b >= 1)
            def _():
                pltpu.make_async_copy(mb, acc_h.at[idx_c], ss).wait()
                pltpu.make_async_copy(wb, acc_w.at[idx_c], ss).wait()
            for k in range(B // 16):
                idx_c[pl.ds(k * 16, 16)] = idx_d[pl.ds(k * 16, 16)]
            base2 = ebase + (b + 2) * B
            if not last:
                @pl.when(b + 2 < NB)
                def _():
                    pltpu.async_copy(src_hbm.at[pl.ds(base2, B)], idx_s, si)
                    pltpu.async_copy(dst_hbm.at[pl.ds(base2, B)], idx_d, si)

            @plsc.parallel_loop(0, B, 1, unroll=4)
            def _(e):
                v = asb[e, :] + adb[e, :]
                v = jnp.where(v >= 0.0, v, 0.2 * v)
                w_v = jnp.exp(v)
                wb[e, :] = w_v
                for h in range(row_w // 16):
                    mb[e, pl.ds(h * 16, 16)] = (
                        hb[e, pl.ds(h * 16, 16)]
                        * jnp.full((16,), w_v[h], _f32))
            c1 = pltpu.async_copy(mb, acc_h.at[idx_c], ss, add=True)
            c2 = pltpu.async_copy(wb, acc_w.at[idx_c], ss, add=True)
            if last:
                c1.wait()
                c2.wait()
            else:
                @pl.when(b + 2 < NB)
                def _():
                    pltpu.make_async_copy(
                        src_hbm.at[pl.ds(base2, B)], idx_s, si).wait()
                    pltpu.make_async_copy(
                        dst_hbm.at[pl.ds(base2, B)], idx_d, si).wait()
                    pltpu.async_copy(as_hbm.at[idx_s], asb, sg)
                    pltpu.async_copy(ad_hbm.at[idx_d], adb, sg)
                    pltpu.async_copy(h_hbm.at[idx_s], hb, sg)

        issue(0, 0)
        issue(1, 1)

        def pair(i, carry):
            stage(2 * i, 0, False)
            stage(2 * i + 1, 1, False)
            return carry

        lax.fori_loop(0, NB // 2, pair, 0)
        stage(NB - 1, 0, True)
        plsc.subcore_barrier()

        pltpu.sync_copy(acc_h.at[pl.ds(nbase, NPT)],
                        ph_hbm.at[c, pl.ds(nbase, NPT)])
        pltpu.sync_copy(acc_w.at[pl.ds(nbase, NPT)],
                        pw_hbm.at[c, pl.ds(nbase, NPT)])

    return body


_edge_call_cache = {}


def _make_edge_call(row_w):
    if row_w in _edge_call_cache:
        return _edge_call_cache[row_w]
    mesh = plsc.VectorSubcoreMesh(core_axis_name="c", subcore_axis_name="s",
                                  num_cores=NC, num_subcores=NS)
    call = pl.kernel(
        _edge_kernel(row_w),
        out_type=[_sds((NC, NP, row_w)), _sds((NC, NP, 16))],
        mesh=mesh,
        compiler_params=pltpu.CompilerParams(use_tc_tiling_on_sc=False),
        scratch_types=(
            [pltpu.VMEM((B,), jnp.int32),
             pltpu.VMEM((B,), jnp.int32),
             pltpu.VMEM((B, 16), _f32),
             pltpu.VMEM((B, 16), _f32),
             pltpu.VMEM((B, row_w), _f32),
             ] * 2
            + [pltpu.VMEM((B,), jnp.int32),
               pltpu.VMEM((B, 16), _f32),
               pltpu.VMEM((B, row_w), _f32),
               pltpu.VMEM_SHARED((NP, row_w), _f32),
               pltpu.VMEM_SHARED((NP, 16), _f32),
               pltpu.SemaphoreType.DMA,
               pltpu.SemaphoreType.DMA,
               pltpu.SemaphoreType.DMA,
               pltpu.SemaphoreType.DMA,
               pltpu.SemaphoreType.DMA,
               ]),
    )
    _edge_call_cache[row_w] = call
    return call


def _k3_body(ph_ref, pw_ref, w2_ref, a2s_ref, a2d_ref,
             h2_ref, as2_ref, ad2_ref):
    acc = ph_ref[0] + ph_ref[1]
    den = pw_ref[0] + pw_ref[1]
    h_idx = lax.broadcasted_iota(jnp.int32, (16, D_IN), 0)
    d_idx = lax.broadcasted_iota(jnp.int32, (16, D_IN), 1)
    st = (d_idx // HID == h_idx).astype(_f32)
    den_rep = jnp.dot(den, st, preferred_element_type=_f32)
    x2 = acc / (den_rep + 1e-16)
    x2 = jnp.where(x2 > 0.0, x2, jnp.exp(jnp.minimum(x2, 0.0)) - 1.0)
    h2 = jnp.dot(x2, w2_ref[...], preferred_element_type=_f32)
    h2_ref[...] = h2
    j_idx = lax.broadcasted_iota(jnp.int32, (OUT, 16), 1)
    p0 = (j_idx == 0).astype(_f32)
    as2_ref[...] = jnp.dot(h2 * a2s_ref[...], p0, preferred_element_type=_f32)
    ad2_ref[...] = jnp.dot(h2 * a2d_ref[...], p0, preferred_element_type=_f32)


def _k3(ph, pw, w2, a2s, a2d):
    return pl.pallas_call(
        _k3_body,
        out_shape=[_sds((NP, OUT)), _sds((NP, 16)), _sds((NP, 16))],
    )(ph, pw, w2, a2s, a2d)


def _k5_body(ph_ref, pw_ref, out_ref):
    acc = ph_ref[0] + ph_ref[1]
    den = pw_ref[0, :, 0:1] + pw_ref[1, :, 0:1]
    out_ref[...] = acc / (den + 1e-16)


def _k5(ph, pw):
    return pl.pallas_call(_k5_body, out_shape=_sds((NP, OUT)))(ph, pw)


def kernel(x, edge_index, W1, a1_src, a1_dst, W2, a2_src, a2_dst):
    src = edge_index[0].astype(jnp.int32)
    dst = edge_index[1].astype(jnp.int32)
    a1s = a1_src.reshape(1, HEADS * HID)
    a1d = a1_dst.reshape(1, HEADS * HID)
    a2s = a2_src.reshape(1, OUT)
    a2d = a2_dst.reshape(1, OUT)
    zrow = jnp.zeros((NP, D_IN), _f32)
    z16 = jnp.zeros((NP, 16), _f32)

    h1, as1, ad1 = _k1(x, W1, a1s, a1d)
    ph1, pw1 = _make_edge_call(D_IN)(h1, as1, ad1, src, dst, zrow, z16)
    h2, as2, ad2 = _k3(ph1, pw1, W2, a2s, a2d)
    ph2, pw2 = _make_edge_call_preload(OUT)(h2, as2, ad2, src, dst, z16, z16)
    return _k5(ph2, pw2)[:N]
